# Initial kernel scaffold; baseline (speedup 1.0000x reference)
import functools

import jax
import jax.numpy as jnp
from jax import lax
from jax.experimental import pallas as pl
from jax.experimental.pallas import tpu as pltpu
from jax.experimental.pallas import tpu_sc as plsc

ROWS = 320000
D = 128
NSEG = 10000
NC = 2
NS = 16
NW = NC * NS
CHUNK = 256
NCHUNKS = ROWS // CHUNK
IPC = CHUNK // 128
SEG_PER_TILE = NSEG // NS


def _sc_partials(y, idx2d):
  mesh = plsc.VectorSubcoreMesh(
      core_axis_name="c", subcore_axis_name="s", num_cores=NC,
      num_subcores=NS)

  @functools.partial(
      pl.kernel,
      mesh=mesh,
      out_type=(
          jax.ShapeDtypeStruct((NC, NSEG, D), jnp.float32),
          jax.ShapeDtypeStruct((NC, NSEG, 16), jnp.float32),
      ),
      scratch_types=[
          pltpu.VMEM((2, CHUNK, D), jnp.float32),
          pltpu.VMEM((2, IPC, 128), jnp.int32),
          pltpu.VMEM((128, 16), jnp.float32),
          pltpu.VMEM((128, 16), jnp.float32),
          pltpu.VMEM_SHARED((NSEG, D), jnp.float32),
          pltpu.VMEM_SHARED((NSEG, 16), jnp.float32),
      ],
  )
  def k(y_hbm, idx_hbm, sums_hbm, cnts_hbm, ybuf, ibuf, ones, zcnt,
        ssum, scnt):
    c = lax.axis_index("c")
    s = lax.axis_index("s")
    w = s * NC + c

    zero16 = jnp.zeros((16,), jnp.float32)
    one16 = jnp.full((16,), 1.0, jnp.float32)

    def fill_small(i, carry):
      ones[i, :] = one16
      zcnt[i, :] = zero16
      return carry

    lax.fori_loop(0, 128, fill_small, 0)

    def fill_y(i, carry):
      for j in range(D // 16):
        ybuf[0, i, pl.ds(16 * j, 16)] = zero16
      return carry

    lax.fori_loop(0, CHUNK, fill_y, 0)

    base = s * SEG_PER_TILE
    pltpu.sync_copy(ybuf.at[0], ssum.at[pl.ds(base, CHUNK)])
    pltpu.sync_copy(ybuf.at[0], ssum.at[pl.ds(base + CHUNK, CHUNK)])
    pltpu.sync_copy(ybuf.at[0, pl.ds(0, SEG_PER_TILE - 2 * CHUNK)],
                    ssum.at[pl.ds(base + 2 * CHUNK, SEG_PER_TILE - 2 * CHUNK)])
    for kk in range(4):
      pltpu.sync_copy(zcnt, scnt.at[pl.ds(base + kk * 128, 128)])
    pltpu.sync_copy(zcnt.at[pl.ds(0, SEG_PER_TILE - 512)],
                    scnt.at[pl.ds(base + 512, SEG_PER_TILE - 512)])
    plsc.subcore_barrier()

    lo = (w * NCHUNKS) // NW
    hi = ((w + 1) * NCHUNKS) // NW

    def chunk_body(i, carry):
      slot = lax.rem(i, 2)
      pltpu.sync_copy(y_hbm.at[pl.ds(i * CHUNK, CHUNK)], ybuf.at[slot])
      pltpu.sync_copy(idx_hbm.at[pl.ds(i * IPC, IPC)], ibuf.at[slot])
      for j in range(IPC):
        pltpu.sync_copy(ybuf.at[slot, pl.ds(j * 128, 128)],
                        ssum.at[ibuf.at[slot, j]], add=True)
        pltpu.sync_copy(ones, scnt.at[ibuf.at[slot, j]], add=True)
      return carry

    lax.fori_loop(lo, hi, chunk_body, 0)
    plsc.subcore_barrier()

    pltpu.sync_copy(ssum.at[pl.ds(base, SEG_PER_TILE)],
                    sums_hbm.at[c, pl.ds(base, SEG_PER_TILE)])
    pltpu.sync_copy(scnt.at[pl.ds(base, SEG_PER_TILE)],
                    cnts_hbm.at[c, pl.ds(base, SEG_PER_TILE)])

  return k(y, idx2d)


def _merge_body(s_ref, c_ref, o_ref):
  sums = s_ref[0] + s_ref[1]
  cnt = c_ref[0, :, 0:1] + c_ref[1, :, 0:1]
  o_ref[...] = sums / jnp.maximum(cnt, 1.0)


def _merge(sums, cnts):
  blk = 2000
  return pl.pallas_call(
      _merge_body,
      grid=(NSEG // blk,),
      in_specs=[
          pl.BlockSpec((NC, blk, D), lambda i: (0, i, 0)),
          pl.BlockSpec((NC, blk, 16), lambda i: (0, i, 0)),
      ],
      out_specs=pl.BlockSpec((blk, D), lambda i: (i, 0)),
      out_shape=jax.ShapeDtypeStruct((NSEG, D), jnp.float32),
  )(sums, cnts)


@jax.jit
def kernel(y, idx):
  idx2d = idx.astype(jnp.int32).reshape(ROWS // 128, 128)
  sums, cnts = _sc_partials(y, idx2d)
  return _merge(sums, cnts)


# speedup vs baseline: 2.5404x; 2.5404x over previous
import functools

import jax
import jax.numpy as jnp
from jax import lax
from jax.experimental import pallas as pl
from jax.experimental.pallas import tpu as pltpu
from jax.experimental.pallas import tpu_sc as plsc

ROWS = 320000
D = 128
NSEG = 10000
NC = 2
NS = 16
NW = NC * NS
CHUNK = 128
NCHUNKS = ROWS // CHUNK

A = [((w * (NSEG // 8)) // NW) * 8 for w in range(NW + 1)]
MAXOWN = 320
ACCROWS = MAXOWN + 8


def _segment_mean(y, idx):
  mesh = plsc.VectorSubcoreMesh(
      core_axis_name="c", subcore_axis_name="s", num_cores=NC,
      num_subcores=NS)

  @functools.partial(
      pl.kernel,
      mesh=mesh,
      out_type=jax.ShapeDtypeStruct((NSEG, D), jnp.float32),
      compiler_params=pltpu.CompilerParams(needs_layout_passes=False),
      scratch_types=[
          pltpu.VMEM((2, CHUNK, D), jnp.float32),
          pltpu.VMEM((2, 1, 128), jnp.int32),
          pltpu.VMEM((ACCROWS, D), jnp.float32),
          pltpu.VMEM((ACCROWS, 16), jnp.float32),
          pltpu.VMEM((16,), jnp.int32),
      ],
  )
  def k(y_hbm, idx_hbm, out_hbm, ybuf, ibuf, acc, cacc, pbuf):
    c = lax.axis_index("c")
    s = lax.axis_index("s")
    w = c * NS + s

    a_lo = lax.div(w * (NSEG // 8), NW) * 8
    a_hi = lax.div((w + 1) * (NSEG // 8), NW) * 8
    own = a_hi - a_lo

    zero16 = jnp.zeros((16,), jnp.float32)

    def zacc(i, carry):
      for g in range(D // 16):
        acc[i, pl.ds(16 * g, 16)] = zero16
      cacc[i, :] = zero16
      return carry

    lax.fori_loop(0, ACCROWS, zacc, 0)

    def search(tgt):
      def bs_body(_, carry):
        lo, hi = carry
        m = lax.div(lo + hi, 2)
        a = lax.min(lax.mul(lax.div(m, 8), 8), jnp.int32(ROWS - 16))
        pltpu.sync_copy(idx_hbm.at[pl.ds(a, 16)], pbuf)
        v16 = pbuf[...]
        off = m - a
        v = jnp.int32(0)
        for kk in range(16):
          v = jnp.where(off == kk, v16[kk], v)
        big = v >= tgt
        live = lo < hi
        return (jnp.where(live, jnp.where(big, lo, m + 1), lo),
                jnp.where(live, jnp.where(big, m, hi), hi))

      res = lax.fori_loop(0, 19, bs_body, (jnp.int32(0), jnp.int32(ROWS)))
      return res[0]

    r_lo = search(a_lo)
    r_hi = search(a_hi)

    clo = lax.div(r_lo, CHUNK)
    chi = lax.div(r_hi + CHUNK - 1, CHUNK)

    lanes = lax.iota(jnp.int32, 16)

    def process(i, slot):
      pltpu.sync_copy(y_hbm.at[pl.ds(i * CHUNK, CHUNK)], ybuf.at[slot])
      pltpu.sync_copy(idx_hbm.at[pl.ds(i * CHUNK, 128)], ibuf.at[slot, 0])

      def row_body(r, carry):
        iv = plsc.load_gather(ibuf.at[slot, 0], [jnp.full((16,), 0) + r])
        loc = iv - a_lo
        ok = (loc >= 0) & (loc < own)
        tr = MAXOWN + (iv & 7)
        rowv = jnp.where(ok, loc, tr)
        for g in range(D // 16):
          yv = ybuf[slot, r, pl.ds(16 * g, 16)]
          plsc.addupdate_scatter(acc, [rowv, 16 * g + lanes], yv)
        one_l = jnp.full((16,), 1.0, jnp.float32)
        plsc.addupdate_scatter(cacc, [rowv, lanes], one_l)
        return carry

      lax.fori_loop(0, CHUNK, row_body, 0)

    def pair_body(p, carry):
      i0 = clo + 2 * p

      @pl.when(i0 < chi)
      def _():
        process(i0, 0)

      @pl.when(i0 + 1 < chi)
      def _():
        process(i0 + 1, 1)

      return carry

    lax.fori_loop(0, lax.div(chi - clo + 1, 2), pair_body, 0)

    def div_row(r, carry):
      rcp = 1.0 / jnp.maximum(cacc[r, :], 1.0)
      for g in range(D // 16):
        acc[r, pl.ds(16 * g, 16)] = acc[r, pl.ds(16 * g, 16)] * rcp
      return carry

    lax.fori_loop(0, MAXOWN, div_row, 0)

    @pl.when(own == 312)
    def _():
      for (o, nr) in ((0, 128), (128, 128), (256, 56)):
        pltpu.sync_copy(acc.at[pl.ds(o, nr)],
                        out_hbm.at[pl.ds(a_lo + o, nr)])

    @pl.when(own == 320)
    def _():
      for (o, nr) in ((0, 128), (128, 128), (256, 64)):
        pltpu.sync_copy(acc.at[pl.ds(o, nr)],
                        out_hbm.at[pl.ds(a_lo + o, nr)])

  return k(y, idx)


@jax.jit
def kernel(y, idx):
  return _segment_mean(y, idx.astype(jnp.int32))
